# baseline (speedup 1.0000x reference)
import functools
import math

import jax
import jax.numpy as jnp
from jax import lax
from jax.experimental import pallas as pl
from jax.experimental.pallas import tpu as pltpu
from jax.experimental.pallas import tpu_sc as plsc

_EMB = 128
_SCALE = math.sqrt(_EMB)
_NC = 2
_NS = 16
_NW = _NC * _NS
_LANES = 16


_NG = 3
_NS_BUF = 4
_LEAD = 2


def _emb_body(tok_hbm, table_hbm, out_hbm, idx_v, *scratch):
    raw = list(scratch[:_NG])
    cooked = list(scratch[_NG:_NG + _NS_BUF])
    s_in = list(scratch[_NG + _NS_BUF:2 * _NG + _NS_BUF])
    s_out = list(scratch[2 * _NG + _NS_BUF:])
    wid = lax.axis_index("s") * _NC + lax.axis_index("c")
    seq, nb = idx_v.shape
    b0 = wid * nb

    pltpu.sync_copy(tok_hbm.at[:, pl.ds(b0, nb)], idx_v)

    def scale(src, dst):
        def row(i, c):
            for j in range(_EMB // _LANES):
                sl = pl.ds(j * _LANES, _LANES)
                dst[i, sl] = src[i, sl] * _SCALE
            return c

        lax.fori_loop(0, nb, row, 0)

    def gather(h, bgi):
        pltpu.async_copy(table_hbm.at[idx_v.at[h]], raw[bgi], s_in[bgi])

    def slot(g, bg, bs, drain, issue):
        if drain:
            bd = (bs + 2) % _NS_BUF
            pltpu.make_async_copy(
                cooked[bd], out_hbm.at[g, pl.ds(b0, nb)], s_out[bd]
            ).wait()
        if issue:
            gather(g + _LEAD, (bg + _LEAD) % _NG)
        pltpu.make_async_copy(
            table_hbm.at[idx_v.at[g]], raw[bg], s_in[bg]
        ).wait()
        scale(raw[bg], cooked[bs])
        pltpu.async_copy(cooked[bs], out_hbm.at[g, pl.ds(b0, nb)], s_out[bs])

    for g in range(_LEAD):
        gather(g, g % _NG)
    for g in range(_LEAD):
        slot(g, g % _NG, g % _NS_BUF, False, True)

    period = 12
    hi = _LEAD + ((seq - 2 * _LEAD) // period) * period

    def outer(u, carry):
        g = _LEAD + period * u
        for k in range(period):
            slot(g + k, (_LEAD + k) % _NG, (_LEAD + k) % _NS_BUF, True, True)
        return carry

    lax.fori_loop(0, (hi - _LEAD) // period, outer, 0)

    for g in range(hi, seq):
        slot(g, g % _NG, g % _NS_BUF, True, g + _LEAD < seq)
    for g in range(seq - 2, seq):
        bs = g % _NS_BUF
        pltpu.make_async_copy(
            cooked[bs], out_hbm.at[g, pl.ds(b0, nb)], s_out[bs]
        ).wait()


def kernel(tokens, table):
    b, l = tokens.shape
    assert b % _NW == 0 and l >= 16
    nb = b // _NW
    tok_t = tokens.T.astype(jnp.int32)

    grid_kernel = functools.partial(
        pl.kernel,
        mesh=plsc.VectorSubcoreMesh(core_axis_name="c", subcore_axis_name="s"),
        out_type=jax.ShapeDtypeStruct((l, b, _EMB), jnp.float32),
        scratch_types=(
            [pltpu.VMEM((l, nb), jnp.int32)]
            + [pltpu.VMEM((nb, _EMB), jnp.float32)
               for _ in range(_NG + _NS_BUF)]
            + [pltpu.SemaphoreType.DMA for _ in range(_NG + _NS_BUF)]
        ),
    )(_emb_body)

    out = grid_kernel(tok_t, table)
    return jnp.transpose(out, (1, 0, 2))
